# baseline (speedup 1.0000x reference)
import dataclasses
import functools

import jax
import jax.numpy as jnp
from jax.experimental import pallas as pl
from jax.experimental.pallas import tpu as pltpu
from jax.experimental.pallas import tpu_sc as plsc

_BLK = 10000
_NSEG = 512
_NWORK = 32
_LANES = 16


def _sc_compiler_params():
    cp = pltpu.CompilerParams(use_tc_tiling_on_sc=False)
    if "needs_layout_passes" in pltpu.CompilerParams.__dataclass_fields__:
        cp = dataclasses.replace(cp, needs_layout_passes=False)
    return cp


def _dot_t(lhs, rhs):
    return jax.lax.dot_general(lhs, rhs, (((1,), (1,)), ((), ())),
                               preferred_element_type=jnp.float32)


def _mlp_kernel(a_ref, b_ref, w1_ref, b1_ref, w2_ref, b2_ref, w3_ref,
                b3_ref, fu_ref):
    w1 = w1_ref[...].astype(jnp.bfloat16)
    w2 = w2_ref[...].astype(jnp.bfloat16)
    w3 = w3_ref[...].T
    b1 = b1_ref[...].T
    b2 = b2_ref[...].T
    x = jnp.concatenate(
        [a_ref[...].astype(jnp.bfloat16), b_ref[...].astype(jnp.bfloat16)],
        axis=1)
    h1 = jax.nn.silu(_dot_t(w1, x) + b1)
    h2 = jax.nn.silu(
        jnp.dot(w2, h1.astype(jnp.bfloat16),
                preferred_element_type=jnp.float32) + b2)
    y = jnp.sum(h2 * w3, axis=0, keepdims=True) + b3_ref[...]
    fu_ref[...] = (jnp.exp(y) * 0.1).reshape(fu_ref.shape)


def _sc_hist_kernel(seg_ref, out_ref, ids_ref, hist_ref, sem):
    rows = seg_ref.shape[0]
    rpw = rows // _NWORK
    extra = rows - rpw * _NWORK
    w = jax.lax.axis_index("c") * 16 + jax.lax.axis_index("s")
    start = jnp.where(w < extra, w * (rpw + 1),
                      extra * (rpw + 1) + (w - extra) * rpw)

    pltpu.async_copy(seg_ref.at[pl.ds(start, rpw)],
                     ids_ref.at[pl.ds(0, rpw)], sem).wait()

    @pl.when(w < extra)
    def _copy_extra():
        pltpu.async_copy(seg_ref.at[pl.ds(start + rpw, 1)],
                         ids_ref.at[pl.ds(rpw, 1)], sem).wait()

    zeros = jnp.zeros((_LANES,), jnp.float32)

    @pl.loop(0, _LANES * _NSEG, step=_LANES)
    def _zero(i):
        hist_ref[pl.ds(i, _LANES)] = zeros

    base = jax.lax.iota(jnp.int32, _LANES) * _NSEG
    ones = jnp.ones((_LANES,), jnp.float32)

    @pl.loop(0, rpw)
    def _accum(j):
        plsc.addupdate_scatter(hist_ref, [ids_ref[j, :] + base], ones)

    @pl.when(w < extra)
    def _accum_extra():
        plsc.addupdate_scatter(hist_ref, [ids_ref[rpw, :] + base], ones)

    pltpu.async_copy(hist_ref, out_ref.at[w], sem).wait()


@jax.jit
def _run(node_feats_raw, segment_ids, FW1, Fb1, FW2, Fb2, FW3, Fb3):
    n, d = node_feats_raw.shape
    assert d == 640 and n % _BLK == 0 and n % _LANES == 0
    num_blocks = n // _BLK

    w1 = FW1
    w2 = FW2
    w3 = FW3
    b1 = Fb1.reshape(1, -1)
    b2 = Fb2.reshape(1, -1)
    b3 = Fb3.reshape(1, 1)

    fu_flat = pl.pallas_call(
        _mlp_kernel,
        grid=(num_blocks,),
        in_specs=[
            pl.BlockSpec((_BLK, 128), lambda i: (i, 0)),
            pl.BlockSpec((_BLK, 128), lambda i: (i, 4)),
            pl.BlockSpec(w1.shape, lambda i: (0, 0)),
            pl.BlockSpec(b1.shape, lambda i: (0, 0)),
            pl.BlockSpec(w2.shape, lambda i: (0, 0)),
            pl.BlockSpec(b2.shape, lambda i: (0, 0)),
            pl.BlockSpec(w3.shape, lambda i: (0, 0)),
            pl.BlockSpec(b3.shape, lambda i: (0, 0)),
        ],
        out_specs=pl.BlockSpec((1, 1, _BLK), lambda i: (i, 0, 0)),
        out_shape=jax.ShapeDtypeStruct((num_blocks, 1, _BLK), jnp.float32),
        compiler_params=pltpu.CompilerParams(
            dimension_semantics=("arbitrary",)),
    )(node_feats_raw, node_feats_raw, w1, b1, w2, b2, w3, b3)

    rows = n // _LANES
    rpw = rows // _NWORK
    seg2d = segment_ids.reshape(rows, _LANES)
    sc_hist = pl.kernel(
        _sc_hist_kernel,
        out_type=jax.ShapeDtypeStruct((_NWORK, _LANES * _NSEG), jnp.float32),
        mesh=plsc.VectorSubcoreMesh(core_axis_name="c", subcore_axis_name="s"),
        scratch_types=[
            pltpu.VMEM((rpw + 1, _LANES), jnp.int32),
            pltpu.VMEM((_LANES * _NSEG,), jnp.float32),
            pltpu.SemaphoreType.DMA,
        ],
        compiler_params=_sc_compiler_params(),
    )
    partials = sc_hist(seg2d)
    cnt = partials.reshape(_NWORK * _LANES, _NSEG).sum(axis=0)
    eu = (0.6 * cnt) / cnt
    return fu_flat.reshape(n, 1), eu


def kernel(node_feats_raw, energy, forces, stress, EW1, Eb1, EW2, Eb2, EW3,
           Eb3, FW1, Fb1, FW2, Fb2, FW3, Fb3, S_uncert, segment_ids):
    fu_col, energy_uncert = _run(node_feats_raw, segment_ids,
                                 FW1, Fb1, FW2, Fb2, FW3, Fb3)
    force_uncert = jnp.broadcast_to(fu_col, (fu_col.shape[0], 3))
    stress_uncert = jnp.full_like(stress, 0.1 / 16)
    return (energy, forces, stress, energy_uncert, force_uncert, stress_uncert)


# speedup vs baseline: 1.0142x; 1.0142x over previous
import dataclasses
import functools

import jax
import jax.numpy as jnp
from jax.experimental import pallas as pl
from jax.experimental.pallas import tpu as pltpu
from jax.experimental.pallas import tpu_sc as plsc

_BLK = 10000
_NSEG = 512
_NWORK = 32
_LANES = 16


def _sc_compiler_params():
    cp = pltpu.CompilerParams(use_tc_tiling_on_sc=False)
    if "needs_layout_passes" in pltpu.CompilerParams.__dataclass_fields__:
        cp = dataclasses.replace(cp, needs_layout_passes=False)
    return cp


def _dot_t(lhs, rhs):
    return jax.lax.dot_general(lhs, rhs, (((1,), (1,)), ((), ())),
                               preferred_element_type=jnp.float32)


def _mlp_kernel(a_ref, b_ref, w1_ref, b1_ref, w2_ref, b2_ref, w3_ref,
                b3_ref, fu_ref):
    w1 = w1_ref[...].astype(jnp.bfloat16)
    w2 = w2_ref[...].astype(jnp.bfloat16)
    w3 = w3_ref[...].T
    b1 = b1_ref[...].T
    b2 = b2_ref[...].T
    x = jnp.concatenate(
        [a_ref[...].astype(jnp.bfloat16), b_ref[...].astype(jnp.bfloat16)],
        axis=1)
    h1 = jax.nn.silu(_dot_t(w1, x) + b1)
    h2 = jax.nn.silu(
        jnp.dot(w2, h1.astype(jnp.bfloat16),
                preferred_element_type=jnp.float32) + b2)
    y = jnp.sum(h2 * w3, axis=0, keepdims=True) + b3_ref[...]
    fu_ref[...] = (jnp.exp(y) * 0.1).reshape(fu_ref.shape)


def _sc_hist_kernel(seg_ref, out_ref, ids_ref, hist_ref, red_ref, sem):
    rows = seg_ref.shape[0]
    rpw = rows // _NWORK
    extra = rows - rpw * _NWORK
    w = jax.lax.axis_index("c") * 16 + jax.lax.axis_index("s")
    start = jnp.where(w < extra, w * (rpw + 1),
                      extra * (rpw + 1) + (w - extra) * rpw)

    pltpu.async_copy(seg_ref.at[pl.ds(start, rpw)],
                     ids_ref.at[pl.ds(0, rpw)], sem).wait()

    @pl.when(w < extra)
    def _copy_extra():
        pltpu.async_copy(seg_ref.at[pl.ds(start + rpw, 1)],
                         ids_ref.at[pl.ds(rpw, 1)], sem).wait()

    zeros = jnp.zeros((_LANES,), jnp.float32)

    @pl.loop(0, _LANES * _NSEG, step=_LANES)
    def _zero(i):
        hist_ref[pl.ds(i, _LANES)] = zeros

    base = jax.lax.iota(jnp.int32, _LANES) * _NSEG
    ones = jnp.ones((_LANES,), jnp.float32)

    @pl.loop(0, rpw)
    def _accum(j):
        plsc.addupdate_scatter(hist_ref, [ids_ref[j, :] + base], ones)

    @pl.when(w < extra)
    def _accum_extra():
        plsc.addupdate_scatter(hist_ref, [ids_ref[rpw, :] + base], ones)

    @pl.loop(0, _NSEG, step=_LANES)
    def _reduce(b):
        acc = hist_ref[pl.ds(b, _LANES)]
        for l in range(1, _LANES):
            acc = acc + hist_ref[pl.ds(l * _NSEG + b, _LANES)]
        red_ref[pl.ds(b, _LANES)] = acc

    pltpu.async_copy(red_ref, out_ref.at[w], sem).wait()


@jax.jit
def _run(node_feats_raw, segment_ids, FW1, Fb1, FW2, Fb2, FW3, Fb3):
    n, d = node_feats_raw.shape
    assert d == 640 and n % _BLK == 0 and n % _LANES == 0
    num_blocks = n // _BLK

    w1 = FW1
    w2 = FW2
    w3 = FW3
    b1 = Fb1.reshape(1, -1)
    b2 = Fb2.reshape(1, -1)
    b3 = Fb3.reshape(1, 1)

    fu_flat = pl.pallas_call(
        _mlp_kernel,
        grid=(num_blocks,),
        in_specs=[
            pl.BlockSpec((_BLK, 128), lambda i: (i, 0)),
            pl.BlockSpec((_BLK, 128), lambda i: (i, 4)),
            pl.BlockSpec(w1.shape, lambda i: (0, 0)),
            pl.BlockSpec(b1.shape, lambda i: (0, 0)),
            pl.BlockSpec(w2.shape, lambda i: (0, 0)),
            pl.BlockSpec(b2.shape, lambda i: (0, 0)),
            pl.BlockSpec(w3.shape, lambda i: (0, 0)),
            pl.BlockSpec(b3.shape, lambda i: (0, 0)),
        ],
        out_specs=pl.BlockSpec((1, 1, _BLK), lambda i: (i, 0, 0)),
        out_shape=jax.ShapeDtypeStruct((num_blocks, 1, _BLK), jnp.float32),
        compiler_params=pltpu.CompilerParams(
            dimension_semantics=("arbitrary",)),
    )(node_feats_raw, node_feats_raw, w1, b1, w2, b2, w3, b3)

    rows = n // _LANES
    rpw = rows // _NWORK
    seg2d = segment_ids.reshape(rows, _LANES)
    sc_hist = pl.kernel(
        _sc_hist_kernel,
        out_type=jax.ShapeDtypeStruct((_NWORK, _NSEG), jnp.float32),
        mesh=plsc.VectorSubcoreMesh(core_axis_name="c", subcore_axis_name="s"),
        scratch_types=[
            pltpu.VMEM((rpw + 1, _LANES), jnp.int32),
            pltpu.VMEM((_LANES * _NSEG,), jnp.float32),
            pltpu.VMEM((_NSEG,), jnp.float32),
            pltpu.SemaphoreType.DMA,
        ],
        compiler_params=_sc_compiler_params(),
    )
    partials = sc_hist(seg2d)
    cnt = partials.sum(axis=0)
    eu = (0.6 * cnt) / cnt
    return fu_flat.reshape(n, 1), eu


def kernel(node_feats_raw, energy, forces, stress, EW1, Eb1, EW2, Eb2, EW3,
           Eb3, FW1, Fb1, FW2, Fb2, FW3, Fb3, S_uncert, segment_ids):
    fu_col, energy_uncert = _run(node_feats_raw, segment_ids,
                                 FW1, Fb1, FW2, Fb2, FW3, Fb3)
    force_uncert = jnp.broadcast_to(fu_col, (fu_col.shape[0], 3))
    stress_uncert = jnp.full_like(stress, 0.1 / 16)
    return (energy, forces, stress, energy_uncert, force_uncert, stress_uncert)


# speedup vs baseline: 1.0397x; 1.0252x over previous
import dataclasses
import functools

import jax
import jax.numpy as jnp
from jax.experimental import pallas as pl
from jax.experimental.pallas import tpu as pltpu
from jax.experimental.pallas import tpu_sc as plsc

_BLK = 10000
_NSEG = 512
_NWORK = 16
_LANES = 16


def _sc_compiler_params():
    cp = pltpu.CompilerParams(use_tc_tiling_on_sc=False)
    if "needs_layout_passes" in pltpu.CompilerParams.__dataclass_fields__:
        cp = dataclasses.replace(cp, needs_layout_passes=False)
    return cp


def _dot_t(lhs, rhs):
    return jax.lax.dot_general(lhs, rhs, (((1,), (1,)), ((), ())),
                               preferred_element_type=jnp.float32)


def _mlp_kernel(a_ref, b_ref, w1_ref, b1_ref, w2_ref, b2_ref, w3_ref,
                b3_ref, fu_ref):
    w1 = w1_ref[...].astype(jnp.bfloat16)
    w2 = w2_ref[...].astype(jnp.bfloat16)
    w3 = w3_ref[...].T
    b1 = b1_ref[...].T
    b2 = b2_ref[...].T
    x = jnp.concatenate(
        [a_ref[...].astype(jnp.bfloat16), b_ref[...].astype(jnp.bfloat16)],
        axis=1)
    h1 = jax.nn.silu(_dot_t(w1, x) + b1)
    h2 = jax.nn.silu(
        jnp.dot(w2, h1.astype(jnp.bfloat16),
                preferred_element_type=jnp.float32) + b2)
    y = jnp.sum(h2 * w3, axis=0, keepdims=True) + b3_ref[...]
    fu_ref[...] = (jnp.exp(y) * 0.1).reshape(fu_ref.shape)


def _sc_hist_kernel(seg_ref, out_ref, ids_ref, hist_ref, red_ref, sem):
    rows = seg_ref.shape[0]
    rpw = rows // _NWORK
    extra = rows - rpw * _NWORK
    w = jax.lax.axis_index("c") * 16 + jax.lax.axis_index("s")
    start = jnp.where(w < extra, w * (rpw + 1),
                      extra * (rpw + 1) + (w - extra) * rpw)

    pltpu.async_copy(seg_ref.at[pl.ds(start, rpw)],
                     ids_ref.at[pl.ds(0, rpw)], sem).wait()

    @pl.when(w < extra)
    def _copy_extra():
        pltpu.async_copy(seg_ref.at[pl.ds(start + rpw, 1)],
                         ids_ref.at[pl.ds(rpw, 1)], sem).wait()

    zeros = jnp.zeros((_LANES,), jnp.float32)

    @pl.loop(0, _LANES * _NSEG, step=_LANES)
    def _zero(i):
        hist_ref[pl.ds(i, _LANES)] = zeros

    base = jax.lax.iota(jnp.int32, _LANES) * _NSEG
    ones = jnp.ones((_LANES,), jnp.float32)

    @pl.loop(0, rpw)
    def _accum(j):
        plsc.addupdate_scatter(hist_ref, [ids_ref[j, :] + base], ones)

    @pl.when(w < extra)
    def _accum_extra():
        plsc.addupdate_scatter(hist_ref, [ids_ref[rpw, :] + base], ones)

    @pl.loop(0, _NSEG, step=_LANES)
    def _reduce(b):
        acc = hist_ref[pl.ds(b, _LANES)]
        for l in range(1, _LANES):
            acc = acc + hist_ref[pl.ds(l * _NSEG + b, _LANES)]
        red_ref[pl.ds(b, _LANES)] = acc

    pltpu.async_copy(red_ref, out_ref.at[w], sem).wait()


@jax.jit
def _run(node_feats_raw, segment_ids, FW1, Fb1, FW2, Fb2, FW3, Fb3):
    n, d = node_feats_raw.shape
    assert d == 640 and n % _BLK == 0 and n % _LANES == 0
    num_blocks = n // _BLK

    w1 = FW1
    w2 = FW2
    w3 = FW3
    b1 = Fb1.reshape(1, -1)
    b2 = Fb2.reshape(1, -1)
    b3 = Fb3.reshape(1, 1)

    fu_flat = pl.pallas_call(
        _mlp_kernel,
        grid=(num_blocks,),
        in_specs=[
            pl.BlockSpec((_BLK, 128), lambda i: (i, 0)),
            pl.BlockSpec((_BLK, 128), lambda i: (i, 4)),
            pl.BlockSpec(w1.shape, lambda i: (0, 0)),
            pl.BlockSpec(b1.shape, lambda i: (0, 0)),
            pl.BlockSpec(w2.shape, lambda i: (0, 0)),
            pl.BlockSpec(b2.shape, lambda i: (0, 0)),
            pl.BlockSpec(w3.shape, lambda i: (0, 0)),
            pl.BlockSpec(b3.shape, lambda i: (0, 0)),
        ],
        out_specs=pl.BlockSpec((1, 1, _BLK), lambda i: (i, 0, 0)),
        out_shape=jax.ShapeDtypeStruct((num_blocks, 1, _BLK), jnp.float32),
        compiler_params=pltpu.CompilerParams(
            dimension_semantics=("arbitrary",)),
    )(node_feats_raw, node_feats_raw, w1, b1, w2, b2, w3, b3)

    rows = n // _LANES
    rpw = rows // _NWORK
    seg2d = segment_ids.reshape(rows, _LANES)
    sc_hist = pl.kernel(
        _sc_hist_kernel,
        out_type=jax.ShapeDtypeStruct((_NWORK, _NSEG), jnp.float32),
        mesh=plsc.VectorSubcoreMesh(core_axis_name="c", subcore_axis_name="s",
                                    num_cores=1),
        scratch_types=[
            pltpu.VMEM((rpw + 1, _LANES), jnp.int32),
            pltpu.VMEM((_LANES * _NSEG,), jnp.float32),
            pltpu.VMEM((_NSEG,), jnp.float32),
            pltpu.SemaphoreType.DMA,
        ],
        compiler_params=_sc_compiler_params(),
    )
    partials = sc_hist(seg2d)
    cnt = partials.sum(axis=0)
    eu = (0.6 * cnt) / cnt
    return fu_flat.reshape(n, 1), eu


def kernel(node_feats_raw, energy, forces, stress, EW1, Eb1, EW2, Eb2, EW3,
           Eb3, FW1, Fb1, FW2, Fb2, FW3, Fb3, S_uncert, segment_ids):
    fu_col, energy_uncert = _run(node_feats_raw, segment_ids,
                                 FW1, Fb1, FW2, Fb2, FW3, Fb3)
    force_uncert = jnp.broadcast_to(fu_col, (fu_col.shape[0], 3))
    stress_uncert = jnp.full_like(stress, 0.1 / 16)
    return (energy, forces, stress, energy_uncert, force_uncert, stress_uncert)
